# baseline (speedup 1.0000x reference)
import functools

import jax
import jax.numpy as jnp
from jax import lax
from jax.experimental import pallas as pl
from jax.experimental.pallas import tpu as pltpu
from jax.experimental.pallas import tpu_sc as plsc

N_ITEMS = 100000
TEXT_DIM = 384
IMAGE_DIM = 512
OUT_DIM = TEXT_DIM + IMAGE_DIM
BATCH = 16384

_info = plsc.get_sparse_core_info()
_NC, _NS = _info.num_cores, _info.num_subcores
_NW = _NC * _NS
_B_PER_W = BATCH // _NW
_CHUNK = 64
_N_CHUNKS = _B_PER_W // _CHUNK


def _sc_gather_concat(idx_hbm, text_hbm, image_hbm, out_hbm,
                      idx_v, buf0, buf1, gsem0, gsem1, wsem0, wsem1):
    wid = lax.axis_index("s") * _NC + lax.axis_index("c")
    base = wid * _B_PER_W
    pltpu.sync_copy(idx_hbm.at[wid], idx_v)
    bufs = (buf0, buf1)
    gsems = (gsem0, gsem1)
    wsems = (wsem0, wsem1)

    def pair_body(p, carry):
        ghs = []
        for b in range(2):
            c = 2 * p + b
            idx_chunk = idx_v.at[c]
            ghs.append((
                pltpu.async_copy(
                    text_hbm.at[idx_chunk],
                    bufs[b].at[:, pl.ds(0, TEXT_DIM)], gsems[b]),
                pltpu.async_copy(
                    image_hbm.at[idx_chunk],
                    bufs[b].at[:, pl.ds(TEXT_DIM, IMAGE_DIM)], gsems[b]),
            ))
        whs = []
        for b in range(2):
            c = 2 * p + b
            ht, hi = ghs[b]
            ht.wait()
            hi.wait()
            row0 = pl.multiple_of(base + c * _CHUNK, _CHUNK)
            whs.append(pltpu.async_copy(
                bufs[b], out_hbm.at[pl.ds(row0, _CHUNK)], wsems[b]))
        for wh in whs:
            wh.wait()
        return carry

    lax.fori_loop(0, _N_CHUNKS // 2, pair_body, 0)


@jax.jit
def _encode(item_idx, text_features, image_features):
    mesh = plsc.VectorSubcoreMesh(core_axis_name="c", subcore_axis_name="s")
    run = functools.partial(
        pl.kernel,
        mesh=mesh,
        out_type=jax.ShapeDtypeStruct((BATCH, OUT_DIM), jnp.float32),
        scratch_types=[
            pltpu.VMEM((_N_CHUNKS, _CHUNK), jnp.int32),
            pltpu.VMEM((_CHUNK, OUT_DIM), jnp.float32),
            pltpu.VMEM((_CHUNK, OUT_DIM), jnp.float32),
            pltpu.SemaphoreType.DMA,
            pltpu.SemaphoreType.DMA,
            pltpu.SemaphoreType.DMA,
            pltpu.SemaphoreType.DMA,
        ],
    )(_sc_gather_concat)
    idx3d = item_idx.astype(jnp.int32).reshape(_NW, _N_CHUNKS, _CHUNK)
    return run(idx3d, text_features, image_features)


def kernel(item_idx, text_features, image_features):
    return _encode(item_idx, text_features, image_features)


# speedup vs baseline: 1.0311x; 1.0311x over previous
import functools

import jax
import jax.numpy as jnp
from jax import lax
from jax.experimental import pallas as pl
from jax.experimental.pallas import tpu as pltpu
from jax.experimental.pallas import tpu_sc as plsc

N_ITEMS = 100000
TEXT_DIM = 384
IMAGE_DIM = 512
OUT_DIM = TEXT_DIM + IMAGE_DIM
BATCH = 16384

_info = plsc.get_sparse_core_info()
_NC, _NS = _info.num_cores, _info.num_subcores
_NW = _NC * _NS
_B_PER_W = BATCH // _NW
_CHUNK = 128
_N_CHUNKS = _B_PER_W // _CHUNK


def _sc_gather_concat(idx_hbm, text_hbm, image_hbm, out_hbm,
                      idx_v, buf, gsem, wsem):
    wid = lax.axis_index("s") * _NC + lax.axis_index("c")
    base = wid * _B_PER_W
    pltpu.sync_copy(idx_hbm.at[wid], idx_v)

    def chunk_body(c, carry):
        idx_chunk = idx_v.at[c]
        ht = pltpu.async_copy(
            text_hbm.at[idx_chunk], buf.at[:, pl.ds(0, TEXT_DIM)], gsem)
        hi = pltpu.async_copy(
            image_hbm.at[idx_chunk], buf.at[:, pl.ds(TEXT_DIM, IMAGE_DIM)],
            gsem)
        ht.wait()
        hi.wait()
        row0 = pl.multiple_of(base + c * _CHUNK, _CHUNK)
        wh = pltpu.async_copy(
            buf, out_hbm.at[pl.ds(row0, _CHUNK)], wsem)
        wh.wait()
        return carry

    lax.fori_loop(0, _N_CHUNKS, chunk_body, 0)


@jax.jit
def _encode(item_idx, text_features, image_features):
    mesh = plsc.VectorSubcoreMesh(core_axis_name="c", subcore_axis_name="s")
    run = functools.partial(
        pl.kernel,
        mesh=mesh,
        out_type=jax.ShapeDtypeStruct((BATCH, OUT_DIM), jnp.float32),
        scratch_types=[
            pltpu.VMEM((_N_CHUNKS, _CHUNK), jnp.int32),
            pltpu.VMEM((_CHUNK, OUT_DIM), jnp.float32),
            pltpu.SemaphoreType.DMA,
            pltpu.SemaphoreType.DMA,
        ],
    )(_sc_gather_concat)
    idx3d = item_idx.astype(jnp.int32).reshape(_NW, _N_CHUNKS, _CHUNK)
    return run(idx3d, text_features, image_features)


def kernel(item_idx, text_features, image_features):
    return _encode(item_idx, text_features, image_features)
